# baseline (speedup 1.0000x reference)
import functools

import jax
import jax.numpy as jnp
from jax import lax
from jax.experimental import pallas as pl
from jax.experimental.pallas import tpu as pltpu
from jax.experimental.pallas import tpu_sc as plsc

N = 10000
NP = 10240
E = 320000
D = 128
NC = 2
NS = 16
CH = 128
CPT = 80
EPAD = NC * NS * CPT * CH
ROWS_PT = NP // NS

_mesh = plsc.VectorSubcoreMesh(core_axis_name="c", subcore_axis_name="s")


def _zero_f32(ref, n):
    z = jnp.zeros((16,), jnp.float32)

    def body(i, _):
        ref[pl.ds(i * 16, 16)] = z
        return 0

    lax.fori_loop(0, n // 16, body, 0)


@functools.partial(
    pl.kernel,
    mesh=_mesh,
    out_type=jax.ShapeDtypeStruct((NC, NS, ROWS_PT), jnp.float32),
    scratch_types=[
        pltpu.VMEM((CPT, CH), jnp.int32),
        pltpu.VMEM((CH,), jnp.float32),
        pltpu.VMEM((ROWS_PT,), jnp.float32),
        pltpu.VMEM_SHARED((NP,), jnp.float32),
        pltpu.SemaphoreType.DMA,
    ],
)
def _deg_kernel(dstp_hbm, out_hbm, didx, ones_v, zbuf, deg_sh, sem):
    c = lax.axis_index("c")
    s = lax.axis_index("s")
    w = c * NS + s

    _zero_f32(zbuf, ROWS_PT)
    o = jnp.ones((16,), jnp.float32)
    for i in range(CH // 16):
        ones_v[pl.ds(i * 16, 16)] = o
    pltpu.sync_copy(dstp_hbm.at[w], didx)
    pltpu.sync_copy(zbuf, deg_sh.at[pl.ds(s * ROWS_PT, ROWS_PT)])
    plsc.subcore_barrier()

    def grp(g, _):
        for k in range(10):
            pltpu.async_copy(ones_v, deg_sh.at[didx.at[g * 10 + k]], sem, add=True)
        for k in range(10):
            pltpu.make_async_copy(ones_v, deg_sh.at[didx.at[g * 10 + k]], sem).wait()
        return 0

    lax.fori_loop(0, CPT // 10, grp, 0)
    plsc.subcore_barrier()
    pltpu.sync_copy(deg_sh.at[pl.ds(s * ROWS_PT, ROWS_PT)], out_hbm.at[c, s])


@functools.partial(
    pl.kernel,
    mesh=_mesh,
    out_type=jax.ShapeDtypeStruct((NC, NS, ROWS_PT, D), jnp.float32),
    scratch_types=[
        pltpu.VMEM((CH,), jnp.int32),
        pltpu.VMEM((CH,), jnp.int32),
        pltpu.VMEM((CH,), jnp.int32),
        pltpu.VMEM((CH,), jnp.int32),
        pltpu.VMEM((CH, D), jnp.float32),
        pltpu.VMEM((CH, D), jnp.float32),
        pltpu.VMEM_SHARED((NP, D), jnp.float32),
        pltpu.SemaphoreType.DMA,
        pltpu.SemaphoreType.DMA,
    ],
)
def _agg_kernel(
    hp_hbm, srcp_hbm, dstp_hbm, out_hbm, sidx_a, sidx_b, didx_a, didx_b,
    rows_a, rows_b, acc_sh, sem_a, sem_b,
):
    c = lax.axis_index("c")
    s = lax.axis_index("s")
    w = c * NS + s
    sidx = (sidx_a, sidx_b)
    didx = (didx_a, didx_b)
    rows = (rows_a, rows_b)
    sems = (sem_a, sem_b)

    z = jnp.zeros((16,), jnp.float32)

    def zrow(r, _):
        for i in range(D // 16):
            rows_a[r, pl.ds(i * 16, 16)] = z
        return 0

    lax.fori_loop(0, CH, zrow, 0)
    for j in range(ROWS_PT // CH):
        pltpu.sync_copy(rows_a, acc_sh.at[pl.ds(s * ROWS_PT + j * CH, CH), :])
    plsc.subcore_barrier()

    def ldidx(j, b):
        pltpu.sync_copy(srcp_hbm.at[w, j], sidx[b])
        pltpu.sync_copy(dstp_hbm.at[w, j], didx[b])

    def gather(b):
        pltpu.make_async_copy(hp_hbm.at[sidx[b]], rows[b], sems[b]).start()

    def gwait(b):
        pltpu.make_async_copy(hp_hbm.at[sidx[b]], rows[b], sems[b]).wait()

    def scat(b):
        pltpu.sync_copy(rows[b], acc_sh.at[didx[b]], add=True)

    ldidx(0, 0)
    gather(0)

    def pair(t, _):
        j0 = 2 * t
        ldidx(j0 + 1, 1)
        gather(1)
        gwait(0)
        scat(0)

        @pl.when(j0 + 2 < CPT)
        def _():
            ldidx(j0 + 2, 0)
            gather(0)

        gwait(1)
        scat(1)
        return 0

    lax.fori_loop(0, CPT // 2, pair, 0)
    plsc.subcore_barrier()
    pltpu.sync_copy(acc_sh.at[pl.ds(s * ROWS_PT, ROWS_PT), :], out_hbm.at[c, s])


def _hprime_body(degp_ref, x_ref, wgt_ref, hp_ref, dis_ref):
    deg = degp_ref[0] + degp_ref[1] + 1.0
    dis = lax.rsqrt(deg)
    dis_ref[...] = dis[:, None]
    h = jnp.dot(x_ref[...], wgt_ref[...], preferred_element_type=jnp.float32)
    hp_ref[...] = h * dis[:, None]


def _final_body(accp_ref, hp_ref, dis_ref, bg_ref, wc_ref, bc_ref, out_ref):
    pre = (accp_ref[0] + accp_ref[1] + hp_ref[...]) * dis_ref[...] + bg_ref[...]
    act = jnp.maximum(pre, 0.0)
    out_ref[...] = (
        jnp.dot(act, wc_ref[...], preferred_element_type=jnp.float32) + bc_ref[...]
    )


def kernel(X, edge_index, W_gcn, b_gcn, W_cls, b_cls):
    src = edge_index[0].astype(jnp.int32)
    dst = edge_index[1].astype(jnp.int32)
    npad = EPAD - E
    srcp = jnp.concatenate([src, jnp.zeros((npad,), jnp.int32)])
    dstp = jnp.concatenate([dst, jnp.full((npad,), N, jnp.int32)])
    srcp = srcp.reshape(NC * NS, CPT, CH)
    dstp = dstp.reshape(NC * NS, CPT, CH)

    degp = _deg_kernel(dstp)
    degp = degp.reshape(NC, NP)

    Xp = jnp.zeros((NP, D), X.dtype).at[:N].set(X)
    RB = 1280
    hp, dis = pl.pallas_call(
        _hprime_body,
        grid=(NP // RB,),
        in_specs=[
            pl.BlockSpec((NC, RB), lambda i: (0, i)),
            pl.BlockSpec((RB, D), lambda i: (i, 0)),
            pl.BlockSpec((D, D), lambda i: (0, 0)),
        ],
        out_specs=[
            pl.BlockSpec((RB, D), lambda i: (i, 0)),
            pl.BlockSpec((RB, 1), lambda i: (i, 0)),
        ],
        out_shape=[
            jax.ShapeDtypeStruct((NP, D), jnp.float32),
            jax.ShapeDtypeStruct((NP, 1), jnp.float32),
        ],
    )(degp, Xp, W_gcn.T)

    accp = _agg_kernel(hp, srcp, dstp)
    accp = accp.reshape(NC, NP, D)

    wc_pad = jnp.zeros((D, D), jnp.float32).at[:, : W_cls.shape[0]].set(W_cls.T)
    bc_pad = jnp.zeros((1, D), jnp.float32).at[0, : W_cls.shape[0]].set(b_cls)

    logits = pl.pallas_call(
        _final_body,
        grid=(NP // RB,),
        in_specs=[
            pl.BlockSpec((NC, RB, D), lambda i: (0, i, 0)),
            pl.BlockSpec((RB, D), lambda i: (i, 0)),
            pl.BlockSpec((RB, 1), lambda i: (i, 0)),
            pl.BlockSpec((1, D), lambda i: (0, 0)),
            pl.BlockSpec((D, D), lambda i: (0, 0)),
            pl.BlockSpec((1, D), lambda i: (0, 0)),
        ],
        out_specs=pl.BlockSpec((RB, D), lambda i: (i, 0)),
        out_shape=jax.ShapeDtypeStruct((NP, D), jnp.float32),
    )(accp, hp, dis, b_gcn.reshape(1, D), wc_pad, bc_pad)

    return logits[:N, : W_cls.shape[0]]


# speedup vs baseline: 1.6303x; 1.6303x over previous
import functools

import jax
import jax.numpy as jnp
from jax import lax
from jax.experimental import pallas as pl
from jax.experimental.pallas import tpu as pltpu
from jax.experimental.pallas import tpu_sc as plsc

N = 10000
NP = 10240
E = 320000
D = 128
NC = 2
NS = 16
CH = 128
NCHUNK = 2528
CPT_DEG = NCHUNK // (NC * NS)
CPT0 = 114
CPT1 = 44
EPAD = NCHUNK * CH
ROWS_PT = NP // NS

_mesh = plsc.VectorSubcoreMesh(core_axis_name="c", subcore_axis_name="s")


def _zero_f32(ref, n):
    z = jnp.zeros((16,), jnp.float32)

    def body(i, _):
        ref[pl.ds(i * 16, 16)] = z
        return 0

    lax.fori_loop(0, n // 16, body, 0)


@functools.partial(
    pl.kernel,
    mesh=_mesh,
    out_type=jax.ShapeDtypeStruct((NC, NS, ROWS_PT), jnp.float32),
    scratch_types=[
        pltpu.VMEM((CPT_DEG, 2, CH), jnp.int32),
        pltpu.VMEM((CH,), jnp.float32),
        pltpu.VMEM((ROWS_PT,), jnp.float32),
        pltpu.VMEM_SHARED((NP,), jnp.float32),
        pltpu.SemaphoreType.DMA,
    ],
)
def _deg_kernel(idx_hbm, out_hbm, idxv, ones_v, zbuf, deg_sh, sem):
    c = lax.axis_index("c")
    s = lax.axis_index("s")
    w = c * NS + s

    _zero_f32(zbuf, ROWS_PT)
    o = jnp.ones((16,), jnp.float32)
    for i in range(CH // 16):
        ones_v[pl.ds(i * 16, 16)] = o
    pltpu.sync_copy(idx_hbm.at[pl.ds(w * CPT_DEG, CPT_DEG)], idxv)
    pltpu.sync_copy(zbuf, deg_sh.at[pl.ds(s * ROWS_PT, ROWS_PT)])
    plsc.subcore_barrier()

    def grp(g, _):
        for k in range(10):
            pltpu.async_copy(ones_v, deg_sh.at[idxv.at[g * 10 + k, 1]], sem, add=True)
        for k in range(10):
            pltpu.make_async_copy(ones_v, deg_sh.at[idxv.at[g * 10 + k, 1]], sem).wait()
        return 0

    ngrp, tail = divmod(CPT_DEG, 10)
    lax.fori_loop(0, ngrp, grp, 0)
    for k in range(tail):
        pltpu.async_copy(ones_v, deg_sh.at[idxv.at[ngrp * 10 + k, 1]], sem, add=True)
    for k in range(tail):
        pltpu.make_async_copy(ones_v, deg_sh.at[idxv.at[ngrp * 10 + k, 1]], sem).wait()
    plsc.subcore_barrier()
    pltpu.sync_copy(deg_sh.at[pl.ds(s * ROWS_PT, ROWS_PT)], out_hbm.at[c, s])


@functools.partial(
    pl.kernel,
    mesh=_mesh,
    out_type=jax.ShapeDtypeStruct((NC, NS, ROWS_PT, D), jnp.float32),
    scratch_types=[
        pltpu.VMEM((2, CH), jnp.int32),
        pltpu.VMEM((2, CH), jnp.int32),
        pltpu.VMEM((CH, D), jnp.float32),
        pltpu.VMEM((CH, D), jnp.float32),
        pltpu.VMEM_SHARED((NP, D), jnp.float32),
        pltpu.SemaphoreType.DMA,
        pltpu.SemaphoreType.DMA,
    ],
)
def _agg_kernel(
    hp_hbm, idx_hbm, out_hbm, idx_a, idx_b, rows_a, rows_b, acc_sh, sem_a, sem_b
):
    c = lax.axis_index("c")
    s = lax.axis_index("s")
    idx = (idx_a, idx_b)
    rows = (rows_a, rows_b)
    sems = (sem_a, sem_b)
    start = jnp.where(c == 0, s * CPT0, NS * CPT0 + s * CPT1)
    npairs = jnp.where(c == 0, CPT0 // 2, CPT1 // 2)

    z = jnp.zeros((16,), jnp.float32)

    def zrow(r, _):
        for i in range(D // 16):
            rows_a[r, pl.ds(i * 16, 16)] = z
        return 0

    lax.fori_loop(0, CH, zrow, 0)
    for j in range(ROWS_PT // CH):
        pltpu.sync_copy(rows_a, acc_sh.at[pl.ds(s * ROWS_PT + j * CH, CH), :])
    plsc.subcore_barrier()

    def ldidx(j, b):
        pltpu.sync_copy(idx_hbm.at[start + j], idx[b])

    def gather(b):
        pltpu.make_async_copy(hp_hbm.at[idx[b].at[0]], rows[b], sems[b]).start()

    def gwait(b):
        pltpu.make_async_copy(hp_hbm.at[idx[b].at[0]], rows[b], sems[b]).wait()

    def scat(b):
        pltpu.sync_copy(rows[b], acc_sh.at[idx[b].at[1]], add=True)

    ldidx(0, 0)
    gather(0)

    def pair(t, _):
        j0 = 2 * t
        ldidx(j0 + 1, 1)
        gather(1)
        gwait(0)
        scat(0)

        @pl.when(t + 1 < npairs)
        def _():
            ldidx(j0 + 2, 0)
            gather(0)

        gwait(1)
        scat(1)
        return 0

    lax.fori_loop(0, npairs, pair, 0)
    plsc.subcore_barrier()
    pltpu.sync_copy(acc_sh.at[pl.ds(s * ROWS_PT, ROWS_PT), :], out_hbm.at[c, s])


def _hprime_body(degp_ref, x_ref, wgt_ref, hp_ref, dis_ref):
    deg = degp_ref[0] + degp_ref[1] + 1.0
    dis = lax.rsqrt(deg)
    dis_ref[...] = dis[:, None]
    h = jnp.dot(x_ref[...], wgt_ref[...], preferred_element_type=jnp.float32)
    hp_ref[...] = h * dis[:, None]


def _final_body(accp_ref, hp_ref, dis_ref, bg_ref, wc_ref, bc_ref, out_ref):
    pre = (accp_ref[0] + accp_ref[1] + hp_ref[...]) * dis_ref[...] + bg_ref[...]
    act = jnp.maximum(pre, 0.0)
    out_ref[...] = (
        jnp.dot(act, wc_ref[...], preferred_element_type=jnp.float32) + bc_ref[...]
    )


def kernel(X, edge_index, W_gcn, b_gcn, W_cls, b_cls):
    src = edge_index[0].astype(jnp.int32)
    dst = edge_index[1].astype(jnp.int32)
    npad = EPAD - E
    srcp = jnp.concatenate([src, jnp.zeros((npad,), jnp.int32)])
    dstp = jnp.concatenate([dst, jnp.full((npad,), N, jnp.int32)])
    idxp = jnp.stack(
        [srcp.reshape(NCHUNK, CH), dstp.reshape(NCHUNK, CH)], axis=1
    )

    degp = _deg_kernel(idxp)
    degp = degp.reshape(NC, NP)

    Xp = jnp.zeros((NP, D), X.dtype).at[:N].set(X)
    RB = 1280
    hp, dis = pl.pallas_call(
        _hprime_body,
        grid=(NP // RB,),
        in_specs=[
            pl.BlockSpec((NC, RB), lambda i: (0, i)),
            pl.BlockSpec((RB, D), lambda i: (i, 0)),
            pl.BlockSpec((D, D), lambda i: (0, 0)),
        ],
        out_specs=[
            pl.BlockSpec((RB, D), lambda i: (i, 0)),
            pl.BlockSpec((RB, 1), lambda i: (i, 0)),
        ],
        out_shape=[
            jax.ShapeDtypeStruct((NP, D), jnp.float32),
            jax.ShapeDtypeStruct((NP, 1), jnp.float32),
        ],
    )(degp, Xp, W_gcn.T)

    accp = _agg_kernel(hp, idxp)
    accp = accp.reshape(NC, NP, D)

    wc_pad = jnp.zeros((D, D), jnp.float32).at[:, : W_cls.shape[0]].set(W_cls.T)
    bc_pad = jnp.zeros((1, D), jnp.float32).at[0, : W_cls.shape[0]].set(b_cls)

    logits = pl.pallas_call(
        _final_body,
        grid=(NP // RB,),
        in_specs=[
            pl.BlockSpec((NC, RB, D), lambda i: (0, i, 0)),
            pl.BlockSpec((RB, D), lambda i: (i, 0)),
            pl.BlockSpec((RB, 1), lambda i: (i, 0)),
            pl.BlockSpec((1, D), lambda i: (0, 0)),
            pl.BlockSpec((D, D), lambda i: (0, 0)),
            pl.BlockSpec((1, D), lambda i: (0, 0)),
        ],
        out_specs=pl.BlockSpec((RB, D), lambda i: (i, 0)),
        out_shape=jax.ShapeDtypeStruct((NP, D), jnp.float32),
    )(accp, hp, dis, b_gcn.reshape(1, D), wc_pad, bc_pad)

    return logits[:N, : W_cls.shape[0]]
